# Initial kernel scaffold; baseline (speedup 1.0000x reference)
import jax
import jax.numpy as jnp
from jax import lax
from jax.experimental import pallas as pl
from jax.experimental.pallas import tpu as pltpu
from jax.experimental.pallas import tpu_sc as plsc

_N = 10000
_E = 160000
_D = 256
_H1 = 32
_H2 = 64
_OUT = 104
_P = 512

_NP = 10240
_DUMMY = _N
_NW = 32
_EC = 128
_EK = 40
_EP = _NW * _EK * _EC
_PC = 64
_PK = 5
_PP = 640
_F3 = 112
_RB = 1024
_NG = _NP // _RB

_mesh = plsc.VectorSubcoreMesh(core_axis_name="c", subcore_axis_name="s")


def _make_sc_pass(feat, n_out, n_chunks, chunk, mode):
    stripe = n_out // 16

    def body(table, src, dst, zeros, out, src_v, dst_v, rows_v, shared, sem):
        c = lax.axis_index("c")
        s = lax.axis_index("s")
        wid = c * 16 + s
        pltpu.sync_copy(zeros.at[pl.ds(s * stripe, stripe)],
                        shared.at[pl.ds(s * stripe, stripe)])
        pltpu.sync_copy(dst.at[wid], dst_v)
        if mode == "gather":
            pltpu.sync_copy(src.at[wid], src_v)
        if mode == "const":
            pltpu.sync_copy(table, rows_v)
        plsc.subcore_barrier()

        @pl.loop(0, n_chunks)
        def _(k):
            if mode == "gather":
                pltpu.async_copy(table.at[src_v.at[k]], rows_v, sem).wait()
            elif mode == "linear":
                base = (wid * n_chunks + k) * chunk
                pltpu.async_copy(table.at[pl.ds(base, chunk)], rows_v, sem).wait()
            pltpu.sync_copy(rows_v, shared.at[dst_v.at[k]], add=True)

        plsc.subcore_barrier()
        pltpu.sync_copy(shared.at[pl.ds(s * stripe, stripe)],
                        out.at[c, pl.ds(s * stripe, stripe)])

    return pl.kernel(
        body,
        out_type=jax.ShapeDtypeStruct((2, n_out, feat), jnp.float32),
        mesh=_mesh,
        scratch_types=[
            pltpu.VMEM((n_chunks, chunk), jnp.int32),
            pltpu.VMEM((n_chunks, chunk), jnp.int32),
            pltpu.VMEM((chunk, feat), jnp.float32),
            pltpu.VMEM_SHARED((n_out, feat), jnp.float32),
            pltpu.SemaphoreType.DMA,
        ],
    )


_sc_deg = _make_sc_pass(16, _NP, _EK, _EC, "const")
_sc_edge32 = _make_sc_pass(_H1, _NP, _EK, _EC, "gather")
_sc_edge64 = _make_sc_pass(_H2, _NP, _EK, _EC, "gather")
_sc_edge112 = _make_sc_pass(_F3, _NP, _EK, _EC, "gather")
_sc_pool = _make_sc_pass(_F3, _PP, _PK, _PC, "linear")


def _tc_mm0(x, w):
    def body(x_ref, w_ref, o_ref):
        o_ref[...] = jnp.dot(x_ref[...], w_ref[...],
                             preferred_element_type=jnp.float32)

    return pl.pallas_call(
        body, grid=(_NG,),
        in_specs=[pl.BlockSpec((_RB, _D), lambda i: (i, 0)),
                  pl.BlockSpec((_D, _H1), lambda i: (0, 0))],
        out_specs=pl.BlockSpec((_RB, _H1), lambda i: (i, 0)),
        out_shape=jax.ShapeDtypeStruct((_NP, _H1), jnp.float32),
    )(x, w)


def _tc_scale0(degp, hw1):
    def body(degp_ref, hw_ref, dinv_ref, h1_ref):
        i = pl.program_id(0)
        deg = degp_ref[0, :, 0:1] + degp_ref[1, :, 0:1] + 1.0
        rows = i * _RB + lax.broadcasted_iota(jnp.int32, (_RB, 1), 0)
        dinv = jnp.where(rows < _N, 1.0 / jnp.sqrt(deg), 0.0)
        dinv_ref[...] = dinv
        h1_ref[...] = hw_ref[...] * dinv

    return pl.pallas_call(
        body, grid=(_NG,),
        in_specs=[pl.BlockSpec((2, _RB, 16), lambda i: (0, i, 0)),
                  pl.BlockSpec((_RB, _H1), lambda i: (i, 0))],
        out_specs=[pl.BlockSpec((_RB, 1), lambda i: (i, 0)),
                   pl.BlockSpec((_RB, _H1), lambda i: (i, 0))],
        out_shape=[jax.ShapeDtypeStruct((_NP, 1), jnp.float32),
                   jax.ShapeDtypeStruct((_NP, _H1), jnp.float32)],
    )(degp, hw1)


def _tc_layer(sp, hp, dinv, w, fin, fout):
    def body(sp_ref, hp_ref, dinv_ref, w_ref, o_ref):
        dv = dinv_ref[...]
        a = (sp_ref[0] + sp_ref[1] + hp_ref[...]) * dv
        a = jnp.maximum(a, 0.0)
        o_ref[...] = jnp.dot(a, w_ref[...],
                             preferred_element_type=jnp.float32) * dv

    return pl.pallas_call(
        body, grid=(_NG,),
        in_specs=[pl.BlockSpec((2, _RB, fin), lambda i: (0, i, 0)),
                  pl.BlockSpec((_RB, fin), lambda i: (i, 0)),
                  pl.BlockSpec((_RB, 1), lambda i: (i, 0)),
                  pl.BlockSpec((fin, fout), lambda i: (0, 0))],
        out_specs=pl.BlockSpec((_RB, fout), lambda i: (i, 0)),
        out_shape=jax.ShapeDtypeStruct((_NP, fout), jnp.float32),
    )(sp, hp, dinv, w)


def _tc_final(sp, hp, dinv):
    def body(sp_ref, hp_ref, dinv_ref, o_ref):
        o_ref[...] = (sp_ref[0] + sp_ref[1] + hp_ref[...]) * dinv_ref[...]

    return pl.pallas_call(
        body, grid=(_NG,),
        in_specs=[pl.BlockSpec((2, _RB, _F3), lambda i: (0, i, 0)),
                  pl.BlockSpec((_RB, _F3), lambda i: (i, 0)),
                  pl.BlockSpec((_RB, 1), lambda i: (i, 0))],
        out_specs=pl.BlockSpec((_RB, _F3), lambda i: (i, 0)),
        out_shape=jax.ShapeDtypeStruct((_NP, _F3), jnp.float32),
    )(sp, hp, dinv)


def _tc_pool_sum(pp):
    def body(pp_ref, o_ref):
        o_ref[...] = pp_ref[0, :_P, :_OUT] + pp_ref[1, :_P, :_OUT]

    return pl.pallas_call(
        body,
        out_shape=jax.ShapeDtypeStruct((_P, _OUT), jnp.float32),
    )(pp)


def kernel(x, edge_index, pool_ids, W1, W2, W3):
    src = edge_index[0]
    dst = edge_index[1]
    epad = jnp.full((_EP - _E,), _DUMMY, jnp.int32)
    src_p = jnp.concatenate([src, epad]).reshape(_NW, _EK, _EC)
    dst_p = jnp.concatenate([dst, epad]).reshape(_NW, _EK, _EC)
    x_p = jnp.pad(x, ((0, _NP - _N), (0, 0)))
    pool_p = jnp.concatenate(
        [pool_ids, jnp.full((_NP - _N,), _P, jnp.int32)]).reshape(_NW, _PK, _PC)
    w3p = jnp.pad(W3, ((0, 0), (0, _F3 - _OUT)))
    ones_rows = jnp.ones((_EC, 16), jnp.float32)
    z16 = jnp.zeros((_NP, 16), jnp.float32)
    z32 = jnp.zeros((_NP, _H1), jnp.float32)
    z64 = jnp.zeros((_NP, _H2), jnp.float32)
    z112 = jnp.zeros((_NP, _F3), jnp.float32)
    zpool = jnp.zeros((_PP, _F3), jnp.float32)

    degp = _sc_deg(ones_rows, dst_p, dst_p, z16)
    hw1 = _tc_mm0(x_p, W1)
    dinv, h1p = _tc_scale0(degp, hw1)
    s1 = _sc_edge32(h1p, src_p, dst_p, z32)
    h2p = _tc_layer(s1, h1p, dinv, W2, _H1, _H2)
    s2 = _sc_edge64(h2p, src_p, dst_p, z64)
    h3p = _tc_layer(s2, h2p, dinv, w3p, _H2, _F3)
    s3 = _sc_edge112(h3p, src_p, dst_p, z112)
    h3 = _tc_final(s3, h3p, dinv)
    poolp = _sc_pool(h3, pool_p, pool_p, zpool)
    return _tc_pool_sum(poolp)


# speedup vs baseline: 9.9909x; 9.9909x over previous
import jax
import jax.numpy as jnp
from jax import lax
from jax.experimental import pallas as pl
from jax.experimental.pallas import tpu as pltpu
from jax.experimental.pallas import tpu_sc as plsc

_N = 10000
_E = 160000
_D = 256
_H1 = 32
_H2 = 64
_OUT = 104
_P = 512

_NP = 10240
_DUMMY = _N
_NW = 32
_EC = 128
_EK = 40
_EP = _NW * _EK * _EC
_PC = 64
_PK = 5
_PP = 640
_F3 = 112
_RB = 1024
_NG = _NP // _RB

_mesh = plsc.VectorSubcoreMesh(core_axis_name="c", subcore_axis_name="s")


def _make_sc_pass(feat, n_out, n_chunks, chunk, mode):
    stripe = n_out // 16

    def body(table, src, dst, zeros, out, src_v, dst_v, rows_v, shared, sem):
        c = lax.axis_index("c")
        s = lax.axis_index("s")
        wid = c * 16 + s
        pltpu.sync_copy(zeros.at[pl.ds(s * stripe, stripe)],
                        shared.at[pl.ds(s * stripe, stripe)])
        pltpu.sync_copy(dst.at[wid], dst_v)
        if mode == "gather":
            pltpu.sync_copy(src.at[wid], src_v)
        if mode == "const":
            pltpu.sync_copy(table, rows_v)
        plsc.subcore_barrier()

        @pl.loop(0, n_chunks)
        def _(k):
            if mode == "gather":
                pltpu.async_copy(table.at[src_v.at[k]], rows_v, sem).wait()
            elif mode == "linear":
                base = (wid * n_chunks + k) * chunk
                pltpu.async_copy(table.at[pl.ds(base, chunk)], rows_v, sem).wait()
            pltpu.sync_copy(rows_v, shared.at[dst_v.at[k]], add=True)

        plsc.subcore_barrier()
        pltpu.sync_copy(shared.at[pl.ds(s * stripe, stripe)],
                        out.at[c, pl.ds(s * stripe, stripe)])

    return pl.kernel(
        body,
        out_type=jax.ShapeDtypeStruct((2, n_out, feat), jnp.float32),
        mesh=_mesh,
        compiler_params=pltpu.CompilerParams(use_tc_tiling_on_sc=False),
        scratch_types=[
            pltpu.VMEM((n_chunks, chunk), jnp.int32),
            pltpu.VMEM((n_chunks, chunk), jnp.int32),
            pltpu.VMEM((chunk, feat), jnp.float32),
            pltpu.VMEM_SHARED((n_out, feat), jnp.float32),
            pltpu.SemaphoreType.DMA,
        ],
    )


_sc_deg = _make_sc_pass(16, _NP, _EK, _EC, "const")
_sc_edge32 = _make_sc_pass(_H1, _NP, _EK, _EC, "gather")
_sc_edge64 = _make_sc_pass(_H2, _NP, _EK, _EC, "gather")
_sc_edge112 = _make_sc_pass(_F3, _NP, _EK, _EC, "gather")
_sc_pool = _make_sc_pass(_F3, _PP, _PK, _PC, "linear")


def _tc_mm0(x, w):
    def body(x_ref, w_ref, o_ref):
        o_ref[...] = jnp.dot(x_ref[...], w_ref[...],
                             preferred_element_type=jnp.float32)

    return pl.pallas_call(
        body, grid=(_NG,),
        in_specs=[pl.BlockSpec((_RB, _D), lambda i: (i, 0)),
                  pl.BlockSpec((_D, _H1), lambda i: (0, 0))],
        out_specs=pl.BlockSpec((_RB, _H1), lambda i: (i, 0)),
        out_shape=jax.ShapeDtypeStruct((_NP, _H1), jnp.float32),
    )(x, w)


def _tc_scale0(degp, hw1):
    def body(degp_ref, hw_ref, dinv_ref, h1_ref):
        i = pl.program_id(0)
        deg = degp_ref[0, :, 0:1] + degp_ref[1, :, 0:1] + 1.0
        rows = i * _RB + lax.broadcasted_iota(jnp.int32, (_RB, 1), 0)
        dinv = jnp.where(rows < _N, 1.0 / jnp.sqrt(deg), 0.0)
        dinv_ref[...] = dinv
        h1_ref[...] = hw_ref[...] * dinv

    return pl.pallas_call(
        body, grid=(_NG,),
        in_specs=[pl.BlockSpec((2, _RB, 16), lambda i: (0, i, 0)),
                  pl.BlockSpec((_RB, _H1), lambda i: (i, 0))],
        out_specs=[pl.BlockSpec((_RB, 1), lambda i: (i, 0)),
                   pl.BlockSpec((_RB, _H1), lambda i: (i, 0))],
        out_shape=[jax.ShapeDtypeStruct((_NP, 1), jnp.float32),
                   jax.ShapeDtypeStruct((_NP, _H1), jnp.float32)],
    )(degp, hw1)


def _tc_layer(sp, hp, dinv, w, fin, fout):
    def body(sp_ref, hp_ref, dinv_ref, w_ref, o_ref):
        dv = dinv_ref[...]
        a = (sp_ref[0] + sp_ref[1] + hp_ref[...]) * dv
        a = jnp.maximum(a, 0.0)
        o_ref[...] = jnp.dot(a, w_ref[...],
                             preferred_element_type=jnp.float32) * dv

    return pl.pallas_call(
        body, grid=(_NG,),
        in_specs=[pl.BlockSpec((2, _RB, fin), lambda i: (0, i, 0)),
                  pl.BlockSpec((_RB, fin), lambda i: (i, 0)),
                  pl.BlockSpec((_RB, 1), lambda i: (i, 0)),
                  pl.BlockSpec((fin, fout), lambda i: (0, 0))],
        out_specs=pl.BlockSpec((_RB, fout), lambda i: (i, 0)),
        out_shape=jax.ShapeDtypeStruct((_NP, fout), jnp.float32),
    )(sp, hp, dinv, w)


def _tc_final(sp, hp, dinv):
    def body(sp_ref, hp_ref, dinv_ref, o_ref):
        o_ref[...] = (sp_ref[0] + sp_ref[1] + hp_ref[...]) * dinv_ref[...]

    return pl.pallas_call(
        body, grid=(_NG,),
        in_specs=[pl.BlockSpec((2, _RB, _F3), lambda i: (0, i, 0)),
                  pl.BlockSpec((_RB, _F3), lambda i: (i, 0)),
                  pl.BlockSpec((_RB, 1), lambda i: (i, 0))],
        out_specs=pl.BlockSpec((_RB, _F3), lambda i: (i, 0)),
        out_shape=jax.ShapeDtypeStruct((_NP, _F3), jnp.float32),
    )(sp, hp, dinv)


def _tc_pool_sum(pp):
    def body(pp_ref, o_ref):
        o_ref[...] = pp_ref[0, :_P, :_OUT] + pp_ref[1, :_P, :_OUT]

    return pl.pallas_call(
        body,
        out_shape=jax.ShapeDtypeStruct((_P, _OUT), jnp.float32),
    )(pp)


def kernel(x, edge_index, pool_ids, W1, W2, W3):
    src = edge_index[0]
    dst = edge_index[1]
    epad = jnp.full((_EP - _E,), _DUMMY, jnp.int32)
    src_p = jnp.concatenate([src, epad]).reshape(_NW, _EK, _EC)
    dst_p = jnp.concatenate([dst, epad]).reshape(_NW, _EK, _EC)
    x_p = jnp.pad(x, ((0, _NP - _N), (0, 0)))
    pool_p = jnp.concatenate(
        [pool_ids, jnp.full((_NP - _N,), _P, jnp.int32)]).reshape(_NW, _PK, _PC)
    w3p = jnp.pad(W3, ((0, 0), (0, _F3 - _OUT)))
    ones_rows = jnp.ones((_EC, 16), jnp.float32)
    z16 = jnp.zeros((_NP, 16), jnp.float32)
    z32 = jnp.zeros((_NP, _H1), jnp.float32)
    z64 = jnp.zeros((_NP, _H2), jnp.float32)
    z112 = jnp.zeros((_NP, _F3), jnp.float32)
    zpool = jnp.zeros((_PP, _F3), jnp.float32)

    degp = _sc_deg(ones_rows, dst_p, dst_p, z16)
    hw1 = _tc_mm0(x_p, W1)
    dinv, h1p = _tc_scale0(degp, hw1)
    s1 = _sc_edge32(h1p, src_p, dst_p, z32)
    h2p = _tc_layer(s1, h1p, dinv, W2, _H1, _H2)
    s2 = _sc_edge64(h2p, src_p, dst_p, z64)
    h3p = _tc_layer(s2, h2p, dinv, w3p, _H2, _F3)
    s3 = _sc_edge112(h3p, src_p, dst_p, z112)
    h3 = _tc_final(s3, h3p, dinv)
    poolp = _sc_pool(h3, pool_p, pool_p, zpool)
    return _tc_pool_sum(poolp)


# speedup vs baseline: 13.3691x; 1.3381x over previous
import jax
import jax.numpy as jnp
from jax import lax
from jax.experimental import pallas as pl
from jax.experimental.pallas import tpu as pltpu
from jax.experimental.pallas import tpu_sc as plsc

_N = 10000
_E = 160000
_D = 256
_H1 = 32
_H2 = 64
_OUT = 104
_P = 512

_NP = 10240
_DUMMY = _N
_NW = 32
_EC = 128
_EK = 40
_EP = _NW * _EK * _EC
_PC = 64
_PK = 5
_PP = 640
_F3 = 112
_RB = 1024
_NG = _NP // _RB

_mesh = plsc.VectorSubcoreMesh(core_axis_name="c", subcore_axis_name="s")


def _make_sc_pass(feat, n_out, n_chunks, chunk, mode):
    stripe = n_out // 16

    def body(table, src, dst, zeros, out, src_v, dst_v, rows_v, shared, sem):
        c = lax.axis_index("c")
        s = lax.axis_index("s")
        wid = c * 16 + s
        pltpu.sync_copy(zeros.at[pl.ds(s * stripe, stripe)],
                        shared.at[pl.ds(s * stripe, stripe)])
        pltpu.sync_copy(dst.at[wid], dst_v)
        if mode == "gather":
            pltpu.sync_copy(src.at[wid], src_v)
        if mode == "const":
            pltpu.sync_copy(table, rows_v)
        plsc.subcore_barrier()

        @pl.loop(0, n_chunks)
        def _(k):
            if mode == "gather":
                pltpu.async_copy(table.at[src_v.at[k]], rows_v, sem).wait()
            elif mode == "linear":
                base = (wid * n_chunks + k) * chunk
                pltpu.async_copy(table.at[pl.ds(base, chunk)], rows_v, sem).wait()
            pltpu.sync_copy(rows_v, shared.at[dst_v.at[k]], add=True)

        plsc.subcore_barrier()
        pltpu.sync_copy(shared.at[pl.ds(s * stripe, stripe)],
                        out.at[c, pl.ds(s * stripe, stripe)])

    return pl.kernel(
        body,
        out_type=jax.ShapeDtypeStruct((2, n_out, feat), jnp.float32),
        mesh=_mesh,
        compiler_params=pltpu.CompilerParams(use_tc_tiling_on_sc=False),
        scratch_types=[
            pltpu.VMEM((n_chunks, chunk), jnp.int32),
            pltpu.VMEM((n_chunks, chunk), jnp.int32),
            pltpu.VMEM((chunk, feat), jnp.float32),
            pltpu.VMEM_SHARED((n_out, feat), jnp.float32),
            pltpu.SemaphoreType.DMA,
        ],
    )


_sc_deg = _make_sc_pass(16, _NP, _EK, _EC, "const")
_sc_edge32 = _make_sc_pass(_H1, _NP, _EK, _EC, "gather")
_sc_edge64 = _make_sc_pass(_H2, _NP, _EK, _EC, "gather")
_sc_pool = _make_sc_pass(_H2, _PP, _PK, _PC, "linear")


def _tc_mm0(x, w):
    def body(x_ref, w_ref, o_ref):
        o_ref[...] = jnp.dot(x_ref[...], w_ref[...],
                             preferred_element_type=jnp.float32)

    return pl.pallas_call(
        body, grid=(_NG,),
        in_specs=[pl.BlockSpec((_RB, _D), lambda i: (i, 0)),
                  pl.BlockSpec((_D, _H1), lambda i: (0, 0))],
        out_specs=pl.BlockSpec((_RB, _H1), lambda i: (i, 0)),
        out_shape=jax.ShapeDtypeStruct((_NP, _H1), jnp.float32),
    )(x, w)


def _tc_scale0(degp, hw1):
    def body(degp_ref, hw_ref, dinv_ref, h1_ref):
        i = pl.program_id(0)
        deg = degp_ref[0, :, 0:1] + degp_ref[1, :, 0:1] + 1.0
        rows = i * _RB + lax.broadcasted_iota(jnp.int32, (_RB, 1), 0)
        dinv = jnp.where(rows < _N, 1.0 / jnp.sqrt(deg), 0.0)
        dinv_ref[...] = dinv
        h1_ref[...] = hw_ref[...] * dinv

    return pl.pallas_call(
        body, grid=(_NG,),
        in_specs=[pl.BlockSpec((2, _RB, 16), lambda i: (0, i, 0)),
                  pl.BlockSpec((_RB, _H1), lambda i: (i, 0))],
        out_specs=[pl.BlockSpec((_RB, 1), lambda i: (i, 0)),
                   pl.BlockSpec((_RB, _H1), lambda i: (i, 0))],
        out_shape=[jax.ShapeDtypeStruct((_NP, 1), jnp.float32),
                   jax.ShapeDtypeStruct((_NP, _H1), jnp.float32)],
    )(degp, hw1)


def _tc_elw1(sp, hp, dinv):
    def body(sp_ref, hp_ref, dinv_ref, o_ref):
        dv = dinv_ref[...]
        a = jnp.maximum((sp_ref[0] + sp_ref[1] + hp_ref[...]) * dv, 0.0)
        o_ref[...] = a * dv

    return pl.pallas_call(
        body, grid=(_NG,),
        in_specs=[pl.BlockSpec((2, _RB, _H1), lambda i: (0, i, 0)),
                  pl.BlockSpec((_RB, _H1), lambda i: (i, 0)),
                  pl.BlockSpec((_RB, 1), lambda i: (i, 0))],
        out_specs=pl.BlockSpec((_RB, _H1), lambda i: (i, 0)),
        out_shape=jax.ShapeDtypeStruct((_NP, _H1), jnp.float32),
    )(sp, hp, dinv)


def _tc_mm2(sp, hp, dinv, w):
    def body(sp_ref, hp_ref, dinv_ref, w_ref, o_ref):
        dv = dinv_ref[...]
        agg = (sp_ref[0] + sp_ref[1] + hp_ref[...]) * dv
        a = jnp.maximum(jnp.dot(agg, w_ref[...],
                                preferred_element_type=jnp.float32), 0.0)
        o_ref[...] = a * dv

    return pl.pallas_call(
        body, grid=(_NG,),
        in_specs=[pl.BlockSpec((2, _RB, _H1), lambda i: (0, i, 0)),
                  pl.BlockSpec((_RB, _H1), lambda i: (i, 0)),
                  pl.BlockSpec((_RB, 1), lambda i: (i, 0)),
                  pl.BlockSpec((_H1, _H2), lambda i: (0, 0))],
        out_specs=pl.BlockSpec((_RB, _H2), lambda i: (i, 0)),
        out_shape=jax.ShapeDtypeStruct((_NP, _H2), jnp.float32),
    )(sp, hp, dinv, w)


def _tc_z(sp, hp, dinv):
    def body(sp_ref, hp_ref, dinv_ref, o_ref):
        o_ref[...] = (sp_ref[0] + sp_ref[1] + hp_ref[...]) * dinv_ref[...]

    return pl.pallas_call(
        body, grid=(_NG,),
        in_specs=[pl.BlockSpec((2, _RB, _H2), lambda i: (0, i, 0)),
                  pl.BlockSpec((_RB, _H2), lambda i: (i, 0)),
                  pl.BlockSpec((_RB, 1), lambda i: (i, 0))],
        out_specs=pl.BlockSpec((_RB, _H2), lambda i: (i, 0)),
        out_shape=jax.ShapeDtypeStruct((_NP, _H2), jnp.float32),
    )(sp, hp, dinv)


def _tc_pool_mm(pp, w3):
    def body(pp_ref, w_ref, o_ref):
        q = pp_ref[0, :_P, :] + pp_ref[1, :_P, :]
        o_ref[...] = jnp.dot(q, w_ref[...], preferred_element_type=jnp.float32)

    return pl.pallas_call(
        body,
        out_shape=jax.ShapeDtypeStruct((_P, _OUT), jnp.float32),
    )(pp, w3)


def kernel(x, edge_index, pool_ids, W1, W2, W3):
    src = edge_index[0]
    dst = edge_index[1]
    epad = jnp.full((_EP - _E,), _DUMMY, jnp.int32)
    src_p = jnp.concatenate([src, epad]).reshape(_NW, _EK, _EC)
    dst_p = jnp.concatenate([dst, epad]).reshape(_NW, _EK, _EC)
    x_p = jnp.pad(x, ((0, _NP - _N), (0, 0)))
    pool_p = jnp.concatenate(
        [pool_ids, jnp.full((_NP - _N,), _P, jnp.int32)]).reshape(_NW, _PK, _PC)
    ones_rows = jnp.ones((_EC, 16), jnp.float32)
    z16 = jnp.zeros((_NP, 16), jnp.float32)
    z32 = jnp.zeros((_NP, _H1), jnp.float32)
    z64 = jnp.zeros((_NP, _H2), jnp.float32)
    zpool = jnp.zeros((_PP, _H2), jnp.float32)

    degp = _sc_deg(ones_rows, dst_p, dst_p, z16)
    hw1 = _tc_mm0(x_p, W1)
    dinv, h1p = _tc_scale0(degp, hw1)
    s1 = _sc_edge32(h1p, src_p, dst_p, z32)
    a1pp = _tc_elw1(s1, h1p, dinv)
    s2 = _sc_edge32(a1pp, src_p, dst_p, z32)
    a2pp = _tc_mm2(s2, a1pp, dinv, W2)
    s3 = _sc_edge64(a2pp, src_p, dst_p, z64)
    z = _tc_z(s3, a2pp, dinv)
    poolp = _sc_pool(z, pool_p, pool_p, zpool)
    return _tc_pool_mm(poolp, W3)
